# baseline (speedup 1.0000x reference)
import functools

import jax
import jax.numpy as jnp
from jax import lax
from jax.experimental import pallas as pl
from jax.experimental.pallas import tpu as pltpu
from jax.experimental.pallas import tpu_sc as plsc

NUM_EMB = 1000000
HALF = NUM_EMB // 2
D = 64
B = 16384
H = 20
TOTAL = B * H

NC = 2
NS = 16
NW = NC * NS

PER_W = TOTAL // NW
CHUNK = 128
NCH = PER_W // CHUNK
NBUF = 4
GROUPS = NCH // NBUF

NTILES = NUM_EMB // 8
NPAIR = NTILES // 2
NT = 16
VL = 16


def _pack_kernel(tab_hbm, out_hbm, a_v, b_v, sem_a, sem_b):
    wid = lax.axis_index("s") * NC + lax.axis_index("c")
    lo = wid * NPAIR // NW
    hi = (wid + 1) * NPAIR // NW
    npp = NT // 2

    def repack(ntiles):
        for i in range(ntiles):
            for s in range(8):
                for l in range(D // VL):
                    b_v[4 * i + s // 2, pl.ds((s % 2) * D + l * VL, VL)] = (
                        a_v[i, s, pl.ds(l * VL, VL)])

    def step(k, carry):
        p0 = lo + k * npp
        np_ = jnp.minimum(hi - p0, npp)

        @pl.when(np_ == npp)
        def _full():
            t = pl.multiple_of(2 * p0, 2)
            pltpu.async_copy(tab_hbm.at[pl.ds(t, NT)], a_v, sem_a).wait()
            repack(NT)
            pltpu.async_copy(
                b_v, out_hbm.at[pl.ds(pl.multiple_of(8 * p0, 8), 4 * NT)],
                sem_b).wait()

        @pl.when(np_ < npp)
        def _partial():
            def one(i2, carry2):
                pp = p0 + i2
                pltpu.async_copy(tab_hbm.at[pl.ds(2 * pp, 2)],
                                 a_v.at[pl.ds(0, 2)], sem_a).wait()
                repack(2)
                pltpu.async_copy(
                    b_v.at[pl.ds(0, 8)],
                    out_hbm.at[pl.ds(pl.multiple_of(8 * pp, 8), 8)],
                    sem_b).wait()
                return carry2

            lax.fori_loop(0, np_, one, 0)

        return carry

    nsteps = (hi - lo + npp - 1) // npp
    lax.fori_loop(0, nsteps, step, 0)


def _emb_kernel(idx_hbm, table_hbm, out_hbm, idx_v, *scr):
    rows = scr[:NBUF]
    sem_idx = scr[NBUF]
    gsem = scr[NBUF + 1:NBUF + 1 + NBUF]
    wsem = scr[NBUF + 1 + NBUF:]
    wid = lax.axis_index("s") * NC + lax.axis_index("c")
    base = wid * PER_W
    pltpu.async_copy(idx_hbm.at[wid], idx_v, sem_idx).wait()

    def gather(c, b):
        pltpu.async_copy(table_hbm.at[idx_v.at[c]], rows[b], gsem[b])

    def wb_start(c, b):
        pltpu.async_copy(rows[b], out_hbm.at[pl.ds(base + c * CHUNK, CHUNK)],
                         wsem[b])

    def drain(sem, buf):
        pltpu.make_async_copy(table_hbm.at[pl.ds(0, CHUNK)], buf, sem).wait()

    for b in range(NBUF):
        gather(b, b)

    def body(step, carry):
        for b in range(NBUF):
            c = step * NBUF + b
            drain(gsem[b], rows[b])
            wb_start(c, b)
            drain(wsem[b], rows[b])
            gather(c + NBUF, b)
        return carry

    lax.fori_loop(0, GROUPS - 1, body, 0)

    for b in range(NBUF):
        c = (GROUPS - 1) * NBUF + b
        drain(gsem[b], rows[b])
        wb_start(c, b)
        drain(wsem[b], rows[b])


@jax.jit
def kernel(input_ids, table):
    mesh = plsc.VectorSubcoreMesh(core_axis_name="c", subcore_axis_name="s")

    table3 = jnp.reshape(table, (NTILES, 8, D))
    pack = functools.partial(
        pl.kernel,
        mesh=mesh,
        out_type=jax.ShapeDtypeStruct((HALF, 2 * D), jnp.float32),
        scratch_types=[
            pltpu.VMEM((NT, 8, D), jnp.float32),
            pltpu.VMEM((4 * NT, 2 * D), jnp.float32),
            pltpu.SemaphoreType.DMA,
            pltpu.SemaphoreType.DMA,
        ],
    )(_pack_kernel)
    packed = pack(table3)
    flat_table = jnp.reshape(packed, (NUM_EMB, D))

    idx = jnp.reshape(input_ids.astype(jnp.int32), (NW, NCH, CHUNK))
    run = functools.partial(
        pl.kernel,
        mesh=mesh,
        out_type=jax.ShapeDtypeStruct((TOTAL, D), jnp.float32),
        scratch_types=(
            [pltpu.VMEM((NCH, CHUNK), jnp.int32)]
            + [pltpu.VMEM((CHUNK, D), jnp.float32) for _ in range(NBUF)]
            + [pltpu.SemaphoreType.DMA] * (1 + 2 * NBUF)
        ),
        compiler_params=pltpu.CompilerParams(use_tc_tiling_on_sc=False),
    )(_emb_kernel)
    out = run(idx, flat_table)
    return jnp.reshape(out, (B, H, D))


# speedup vs baseline: 1.1173x; 1.1173x over previous
import functools

import jax
import jax.numpy as jnp
from jax import lax
from jax.experimental import pallas as pl
from jax.experimental.pallas import tpu as pltpu
from jax.experimental.pallas import tpu_sc as plsc

NUM_EMB = 1000000
HALF = NUM_EMB // 2
D = 64
B = 16384
H = 20
TOTAL = B * H

NC = 2
NS = 16
NW = NC * NS

PER_W = TOTAL // NW
CHUNK = 128
NCH = PER_W // CHUNK
NBUF = 4
GROUPS = NCH // NBUF

NTILES = NUM_EMB // 8
NPAIR = NTILES // 2
NT = 16
VL = 16


def _pack_kernel(tab_hbm, out_hbm, a0, a1, b0, b1, sa0, sa1, sb0, sb1):
    wid = lax.axis_index("s") * NC + lax.axis_index("c")
    lo = wid * NPAIR // NW
    hi = (wid + 1) * NPAIR // NW
    npp = NT // 2
    abuf = (a0, a1)
    bbuf = (b0, b1)
    sas = (sa0, sa1)
    sbs = (sb0, sb1)

    def fetch(p0, n, buf, sem):
        pltpu.async_copy(
            tab_hbm.at[pl.ds(pl.multiple_of(16 * p0, 8), 16 * n)],
            buf.at[pl.ds(0, 16 * n)], sem)

    def repack(npairs, a_v, b_v):
        for i in range(2 * npairs):
            for s in range(8):
                for l in range(D // VL):
                    b_v[4 * i + s // 2, pl.ds((s % 2) * D + l * VL, VL)] = (
                        a_v[8 * i + s, pl.ds(l * VL, VL)])

    def wb(p0, n, buf, sem):
        pltpu.async_copy(
            buf.at[pl.ds(0, 8 * n)],
            out_hbm.at[pl.ds(pl.multiple_of(8 * p0, 8), 8 * n)], sem)

    nfull = (hi - lo) // npp

    @pl.when(nfull > 0)
    def _prime():
        fetch(lo, npp, abuf[0], sas[0])

    def step2(m, carry):
        for d in range(2):
            k = 2 * m + d

            @pl.when(k < nfull)
            def _do(k=k, d=d):
                @pl.when(k + 1 < nfull)
                def _pref():
                    fetch(lo + (k + 1) * npp, npp, abuf[1 - d], sas[1 - d])
                pltpu.make_async_copy(
                    tab_hbm.at[pl.ds(0, 16 * npp)],
                    abuf[d].at[pl.ds(0, 16 * npp)], sas[d]).wait()
                @pl.when(k >= 2)
                def _wbw():
                    pltpu.make_async_copy(
                        tab_hbm.at[pl.ds(0, 8 * npp)],
                        bbuf[d].at[pl.ds(0, 8 * npp)], sbs[d]).wait()
                repack(npp, abuf[d], bbuf[d])
                wb(lo + k * npp, npp, bbuf[d], sbs[d])

        return carry

    lax.fori_loop(0, (nfull + 1) // 2, step2, 0)

    @pl.when(nfull > 0)
    def _dr0():
        pltpu.make_async_copy(
            tab_hbm.at[pl.ds(0, 8 * npp)],
            bbuf[0].at[pl.ds(0, 8 * npp)], sbs[0]).wait()

    @pl.when(nfull > 1)
    def _dr1():
        pltpu.make_async_copy(
            tab_hbm.at[pl.ds(0, 8 * npp)],
            bbuf[1].at[pl.ds(0, 8 * npp)], sbs[1]).wait()

    def one(i2, carry2):
        pp = lo + nfull * npp + i2
        fetch(pp, 1, abuf[0], sas[0])
        pltpu.make_async_copy(
            tab_hbm.at[pl.ds(0, 16)], abuf[0].at[pl.ds(0, 16)], sas[0]).wait()
        repack(1, abuf[0], bbuf[0])
        wb(pp, 1, bbuf[0], sbs[0])
        pltpu.make_async_copy(
            tab_hbm.at[pl.ds(0, 8)], bbuf[0].at[pl.ds(0, 8)], sbs[0]).wait()
        return carry2

    lax.fori_loop(0, hi - lo - nfull * npp, one, 0)


def _emb_kernel(idx_hbm, table_hbm, out_hbm, idx_v, *scr):
    rows = scr[:NBUF]
    sem_idx = scr[NBUF]
    gsem = scr[NBUF + 1:NBUF + 1 + NBUF]
    wsem = scr[NBUF + 1 + NBUF:]
    wid = lax.axis_index("s") * NC + lax.axis_index("c")
    base = wid * PER_W
    pltpu.async_copy(idx_hbm.at[wid], idx_v, sem_idx).wait()

    def gather(c, b):
        pltpu.async_copy(table_hbm.at[idx_v.at[c]], rows[b], gsem[b])

    def wb_start(c, b):
        pltpu.async_copy(rows[b], out_hbm.at[pl.ds(base + c * CHUNK, CHUNK)],
                         wsem[b])

    def drain(sem, buf):
        pltpu.make_async_copy(table_hbm.at[pl.ds(0, CHUNK)], buf, sem).wait()

    for b in range(NBUF):
        gather(b, b)

    def body(step, carry):
        for b in range(NBUF):
            c = step * NBUF + b
            drain(gsem[b], rows[b])
            wb_start(c, b)
            drain(wsem[b], rows[b])
            gather(c + NBUF, b)
        return carry

    lax.fori_loop(0, GROUPS - 1, body, 0)

    for b in range(NBUF):
        c = (GROUPS - 1) * NBUF + b
        drain(gsem[b], rows[b])
        wb_start(c, b)
        drain(wsem[b], rows[b])


@jax.jit
def kernel(input_ids, table):
    mesh = plsc.VectorSubcoreMesh(core_axis_name="c", subcore_axis_name="s")

    pack = functools.partial(
        pl.kernel,
        mesh=mesh,
        out_type=jax.ShapeDtypeStruct((HALF, 2 * D), jnp.float32),
        scratch_types=(
            [pltpu.VMEM((8 * NT, D), jnp.float32)] * 2
            + [pltpu.VMEM((4 * NT, 2 * D), jnp.float32)] * 2
            + [pltpu.SemaphoreType.DMA] * 4
        ),
    )(_pack_kernel)
    packed = pack(table)
    flat_table = jnp.reshape(packed, (NUM_EMB, D))

    idx = jnp.reshape(input_ids.astype(jnp.int32), (NW, NCH, CHUNK))
    run = functools.partial(
        pl.kernel,
        mesh=mesh,
        out_type=jax.ShapeDtypeStruct((TOTAL, D), jnp.float32),
        scratch_types=(
            [pltpu.VMEM((NCH, CHUNK), jnp.int32)]
            + [pltpu.VMEM((CHUNK, D), jnp.float32) for _ in range(NBUF)]
            + [pltpu.SemaphoreType.DMA] * (1 + 2 * NBUF)
        ),
        compiler_params=pltpu.CompilerParams(use_tc_tiling_on_sc=False),
    )(_emb_kernel)
    out = run(idx, flat_table)
    return jnp.reshape(out, (B, H, D))


# speedup vs baseline: 1.1413x; 1.0215x over previous
import functools

import jax
import jax.numpy as jnp
from jax import lax
from jax.experimental import pallas as pl
from jax.experimental.pallas import tpu as pltpu
from jax.experimental.pallas import tpu_sc as plsc

NUM_EMB = 1000000
HALF = NUM_EMB // 2
D = 64
B = 16384
H = 20
TOTAL = B * H

NC = 2
NS = 16
NW = NC * NS

PER_W = TOTAL // NW
CHUNK = 128
NCH = PER_W // CHUNK
NBUF = 4
GROUPS = NCH // NBUF

NTILES = NUM_EMB // 8
NPAIR = NTILES // 2
NT = 16
VL = 16


def _pack_kernel(tab_hbm, out_hbm, a0, a1, b0, b1, sa0, sa1, sb0, sb1):
    wid = lax.axis_index("s") * NC + lax.axis_index("c")
    lo = wid * NPAIR // NW
    hi = (wid + 1) * NPAIR // NW
    npp = NT // 2
    abuf = (a0, a1)
    bbuf = (b0, b1)
    sas = (sa0, sa1)
    sbs = (sb0, sb1)

    def fetch(p0, n, buf, sem):
        pltpu.async_copy(
            tab_hbm.at[pl.ds(pl.multiple_of(16 * p0, 8), 16 * n)],
            buf.at[pl.ds(0, 16 * n)], sem)

    def repack(npairs, a_v, b_v):
        for i in range(2 * npairs):
            for s in range(8):
                for l in range(D // VL):
                    b_v[4 * i + s // 2, pl.ds((s % 2) * D + l * VL, VL)] = (
                        a_v[8 * i + s, pl.ds(l * VL, VL)])

    def wb(p0, n, buf, sem):
        pltpu.async_copy(
            buf.at[pl.ds(0, 8 * n)],
            out_hbm.at[pl.ds(pl.multiple_of(8 * p0, 8), 8 * n)], sem)

    nfull = (hi - lo) // npp

    @pl.when(nfull > 0)
    def _prime():
        fetch(lo, npp, abuf[0], sas[0])

    def step2(m, carry):
        for d in range(2):
            k = 2 * m + d

            @pl.when(k < nfull)
            def _do(k=k, d=d):
                @pl.when(k + 1 < nfull)
                def _pref():
                    fetch(lo + (k + 1) * npp, npp, abuf[1 - d], sas[1 - d])
                pltpu.make_async_copy(
                    tab_hbm.at[pl.ds(0, 16 * npp)],
                    abuf[d].at[pl.ds(0, 16 * npp)], sas[d]).wait()
                @pl.when(k >= 2)
                def _wbw():
                    pltpu.make_async_copy(
                        tab_hbm.at[pl.ds(0, 8 * npp)],
                        bbuf[d].at[pl.ds(0, 8 * npp)], sbs[d]).wait()
                repack(npp, abuf[d], bbuf[d])
                wb(lo + k * npp, npp, bbuf[d], sbs[d])

        return carry

    lax.fori_loop(0, (nfull + 1) // 2, step2, 0)

    @pl.when(nfull > 0)
    def _dr0():
        pltpu.make_async_copy(
            tab_hbm.at[pl.ds(0, 8 * npp)],
            bbuf[0].at[pl.ds(0, 8 * npp)], sbs[0]).wait()

    @pl.when(nfull > 1)
    def _dr1():
        pltpu.make_async_copy(
            tab_hbm.at[pl.ds(0, 8 * npp)],
            bbuf[1].at[pl.ds(0, 8 * npp)], sbs[1]).wait()

    def one(i2, carry2):
        pp = lo + nfull * npp + i2
        fetch(pp, 1, abuf[0], sas[0])
        pltpu.make_async_copy(
            tab_hbm.at[pl.ds(0, 16)], abuf[0].at[pl.ds(0, 16)], sas[0]).wait()
        repack(1, abuf[0], bbuf[0])
        wb(pp, 1, bbuf[0], sbs[0])
        pltpu.make_async_copy(
            tab_hbm.at[pl.ds(0, 8)], bbuf[0].at[pl.ds(0, 8)], sbs[0]).wait()
        return carry2

    lax.fori_loop(0, hi - lo - nfull * npp, one, 0)


FB = 256


def _fin_kernel(x_ref, o_ref):
    x = x_ref[...]
    a = jnp.reshape(x[:, :D], (FB, H // 2, D))
    b = jnp.reshape(x[:, D:], (FB, H // 2, D))
    o_ref[...] = jnp.reshape(jnp.stack([a, b], axis=2), (FB, H, D))


def _emb_kernel(idx_hbm, table_hbm, out_hbm, idx_v, *scr):
    rows = scr[:NBUF]
    sem_idx = scr[NBUF]
    gsem = scr[NBUF + 1:NBUF + 1 + NBUF]
    wsem = scr[NBUF + 1 + NBUF:]
    wid = lax.axis_index("s") * NC + lax.axis_index("c")
    base = wid * PER_W
    pltpu.async_copy(idx_hbm.at[wid], idx_v, sem_idx).wait()

    def gather(c, b):
        pltpu.async_copy(table_hbm.at[idx_v.at[c]], rows[b], gsem[b])

    def wb_start(c, b):
        pltpu.async_copy(rows[b], out_hbm.at[pl.ds(base + c * CHUNK, CHUNK)],
                         wsem[b])

    def drain(sem, buf):
        pltpu.make_async_copy(table_hbm.at[pl.ds(0, CHUNK)], buf, sem).wait()

    for b in range(NBUF):
        gather(b, b)

    def body(step, carry):
        for b in range(NBUF):
            c = step * NBUF + b
            drain(gsem[b], rows[b])
            wb_start(c, b)
            drain(wsem[b], rows[b])
            gather(c + NBUF, b)
        return carry

    lax.fori_loop(0, GROUPS - 1, body, 0)

    for b in range(NBUF):
        c = (GROUPS - 1) * NBUF + b
        drain(gsem[b], rows[b])
        wb_start(c, b)
        drain(wsem[b], rows[b])


@jax.jit
def kernel(input_ids, table):
    mesh = plsc.VectorSubcoreMesh(core_axis_name="c", subcore_axis_name="s")

    packed = jnp.concatenate([table[:HALF], table[HALF:]], axis=1)
    flat_table = jnp.reshape(packed, (NUM_EMB, D))

    ids = input_ids.astype(jnp.int32)
    q = jnp.where(ids < HALF, 2 * ids, 2 * ids - (NUM_EMB - 1))
    idx = jnp.reshape(q, (NW, NCH, CHUNK))
    run = functools.partial(
        pl.kernel,
        mesh=mesh,
        out_type=jax.ShapeDtypeStruct((TOTAL, D), jnp.float32),
        scratch_types=(
            [pltpu.VMEM((NCH, CHUNK), jnp.int32)]
            + [pltpu.VMEM((CHUNK, D), jnp.float32) for _ in range(NBUF)]
            + [pltpu.SemaphoreType.DMA] * (1 + 2 * NBUF)
        ),
        compiler_params=pltpu.CompilerParams(use_tc_tiling_on_sc=False),
    )(_emb_kernel)
    out = run(idx, flat_table)

    wide = jnp.reshape(out, (TOTAL // 2, 2 * D))
    return pl.pallas_call(
        _fin_kernel,
        grid=(B // FB,),
        in_specs=[pl.BlockSpec((FB * H // 2, 2 * D), lambda i: (i, 0))],
        out_specs=pl.BlockSpec((FB, H, D), lambda i: (i, 0, 0)),
        out_shape=jax.ShapeDtypeStruct((B, H, D), jnp.float32),
    )(wide)


# speedup vs baseline: 1.2813x; 1.1227x over previous
import functools

import jax
import jax.numpy as jnp
from jax import lax
from jax.experimental import pallas as pl
from jax.experimental.pallas import tpu as pltpu
from jax.experimental.pallas import tpu_sc as plsc

NUM_EMB = 1000000
HALF = NUM_EMB // 2
D = 64
B = 16384
H = 20
TOTAL = B * H

NC = 2
NS = 16
NW = NC * NS
PER_W = TOTAL // NW
CHUNK = 128
NCH = PER_W // CHUNK
NBUF = 4
GROUPS = NCH // NBUF


def _emb_kernel(idx_hbm, table_hbm, out_hbm, idx_v, *scr):
    rows = scr[:NBUF]
    sem_idx = scr[NBUF]
    gsem = scr[NBUF + 1:NBUF + 1 + NBUF]
    wsem = scr[NBUF + 1 + NBUF:]
    wid = lax.axis_index("s") * NC + lax.axis_index("c")
    base = wid * PER_W
    pltpu.async_copy(idx_hbm.at[wid], idx_v, sem_idx).wait()

    def gather(c, b):
        pltpu.async_copy(table_hbm.at[idx_v.at[c]], rows[b], gsem[b])

    def wb_start(c, b):
        pltpu.async_copy(rows[b], out_hbm.at[pl.ds(base + c * CHUNK, CHUNK)],
                         wsem[b])

    def drain(sem, buf):
        pltpu.make_async_copy(table_hbm.at[pl.ds(0, CHUNK)], buf, sem).wait()

    for b in range(NBUF):
        gather(b, b)

    def body(step, carry):
        for b in range(NBUF):
            c = step * NBUF + b
            drain(gsem[b], rows[b])
            wb_start(c, b)
            drain(wsem[b], rows[b])
            gather(c + NBUF, b)
        return carry

    lax.fori_loop(0, GROUPS - 1, body, 0)

    for b in range(NBUF):
        c = (GROUPS - 1) * NBUF + b
        drain(gsem[b], rows[b])
        wb_start(c, b)
        drain(wsem[b], rows[b])


@jax.jit
def kernel(input_ids, table):
    packed = jnp.concatenate([table[:HALF], table[HALF:]], axis=1)
    flat_table = jnp.reshape(packed, (NUM_EMB, D))

    ids = input_ids.astype(jnp.int32)
    q = jnp.where(ids < HALF, 2 * ids, 2 * ids - (NUM_EMB - 1))
    idx = jnp.reshape(q, (NW, NCH, CHUNK))

    mesh = plsc.VectorSubcoreMesh(core_axis_name="c", subcore_axis_name="s")
    run = functools.partial(
        pl.kernel,
        mesh=mesh,
        out_type=jax.ShapeDtypeStruct((TOTAL, D), jnp.float32),
        scratch_types=(
            [pltpu.VMEM((NCH, CHUNK), jnp.int32)]
            + [pltpu.VMEM((CHUNK, D), jnp.float32) for _ in range(NBUF)]
            + [pltpu.SemaphoreType.DMA] * (1 + 2 * NBUF)
        ),
        compiler_params=pltpu.CompilerParams(use_tc_tiling_on_sc=False),
    )(_emb_kernel)
    out = run(idx, flat_table)
    return jnp.reshape(out, (B, H, D))
